# baseline (speedup 1.0000x reference)
import jax
import jax.numpy as jnp
from jax import lax
from jax.experimental import pallas as pl
from jax.experimental.pallas import tpu as pltpu

N_E = 1024
D = 256
BETA = 0.25
CB = 16


def _vq_kernel(x_ref, e_ref, q_ref, idx_ref, loss_ref, e2_ref):
    pid = pl.program_id(0)
    ew = e_ref[...]

    @pl.when(pid == 0)
    def _():
        ones_col = jnp.ones((D, 1), jnp.float32)
        e2_ref[...] = lax.dot_general(
            ones_col, ew * ew, (((0,), (1,)), ((), ())),
            preferred_element_type=jnp.float32)

    e2 = e2_ref[...]
    part = jnp.zeros((1, 1), jnp.float32)
    for k in range(CB):
        xk = x_ref[k]
        mm = lax.dot_general(xk, ew, (((0,), (1,)), ((), ())),
                             preferred_element_type=jnp.float32)
        x2 = lax.dot_general(xk * xk, jnp.ones((D, 1), jnp.float32),
                             (((0,), (0,)), ((), ())),
                             preferred_element_type=jnp.float32)
        d = (x2 + e2) - 2.0 * mm
        dmin = jnp.min(d, axis=1, keepdims=True)
        iota_j = lax.broadcasted_iota(jnp.int32, d.shape, 1)
        idx = jnp.min(jnp.where(d == dmin, iota_j, jnp.int32(2**30)), axis=1)
        idx_ref[k, 0, :] = idx
        onehot = (lax.broadcasted_iota(jnp.int32, (N_E, 256), 0)
                  == idx[None, :]).astype(jnp.float32)
        q_ref[k, :, :] = lax.dot_general(ew, onehot, (((0,), (0,)), ((), ())),
                                         preferred_element_type=jnp.float32)
        part = part + jnp.sum(dmin, axis=(0, 1), keepdims=True)

    @pl.when(pid == 0)
    def _():
        loss_ref[...] = part

    @pl.when(pid != 0)
    def _():
        loss_ref[...] = loss_ref[...] + part


def kernel(inputs, embed_weight):
    input_shape = inputs.shape
    bb, cc, hh, ww = input_shape
    n = inputs.size
    nslab = bb * hh // 8
    xt = jnp.transpose(inputs, (0, 2, 3, 1))
    x3 = xt.reshape(nslab, 8 * ww, cc)
    x3 = lax.optimization_barrier(x3)
    grid = nslab // CB

    q, idx3, lsum = pl.pallas_call(
        _vq_kernel,
        grid=(grid,),
        in_specs=[
            pl.BlockSpec((CB, 8 * ww, cc), lambda i: (i, 0, 0)),
            pl.BlockSpec((N_E, D), lambda i: (0, 0)),
        ],
        out_specs=[
            pl.BlockSpec((CB, 8 * ww, cc), lambda i: (i, 0, 0)),
            pl.BlockSpec((CB, 1, cc), lambda i: (i, 0, 0)),
            pl.BlockSpec((1, 1), lambda i: (0, 0)),
        ],
        out_shape=[
            jax.ShapeDtypeStruct((nslab, 8 * ww, cc), jnp.float32),
            jax.ShapeDtypeStruct((nslab, 1, cc), jnp.int32),
            jax.ShapeDtypeStruct((1, 1), jnp.float32),
        ],
        scratch_shapes=[pltpu.VMEM((1, N_E), jnp.float32)],
        compiler_params=pltpu.CompilerParams(
            dimension_semantics=("arbitrary",)),
    )(x3, embed_weight)

    q4 = q.reshape(bb, hh, ww, cc)
    quantized = jnp.transpose(q4, (0, 3, 1, 2))
    loss = (lsum * jnp.float32((1.0 + BETA) / n)).reshape(())
    idxg = idx3.reshape(bb, hh // 8, cc)
    idx2d = jnp.transpose(idxg, (0, 2, 1)).reshape(bb, hh, ww)
    return (quantized, loss, idx2d)


# speedup vs baseline: 1.0261x; 1.0261x over previous
import jax
import jax.numpy as jnp
from jax import lax
from jax.experimental import pallas as pl
from jax.experimental.pallas import tpu as pltpu

N_E = 1024
D = 256
BETA = 0.25
CB = 8


def _vq_kernel(x_ref, e_ref, q_ref, idx_ref, loss_ref, e2_ref, em2_ref):
    pid = pl.program_id(0)
    ew = e_ref[...]

    @pl.when(pid == 0)
    def _():
        ones_col = jnp.ones((D, 1), jnp.float32)
        e2_ref[...] = lax.dot_general(
            ones_col, ew * ew, (((0,), (1,)), ((), ())),
            preferred_element_type=jnp.float32)
        em2_ref[...] = -2.0 * ew

    e2 = e2_ref[...]
    em2 = em2_ref[...]
    part = jnp.zeros((1, 1), jnp.float32)
    for k in range(CB):
        xk = x_ref[k]
        mm = lax.dot_general(xk.astype(jnp.bfloat16), em2,
                             (((0,), (1,)), ((), ())),
                             preferred_element_type=jnp.float32)
        x2 = lax.dot_general(xk * xk, jnp.ones((D, 1), jnp.float32),
                             (((0,), (0,)), ((), ())),
                             preferred_element_type=jnp.float32)
        d = (x2 + e2) + mm
        dmin = jnp.min(d, axis=1, keepdims=True)
        iota_j = lax.broadcasted_iota(jnp.int32, d.shape, 1)
        idx = jnp.min(jnp.where(d == dmin, iota_j, jnp.int32(2**30)), axis=1)
        idx_ref[k, 0, :] = idx
        onehot = (lax.broadcasted_iota(jnp.int32, (N_E, 256), 0)
                  == idx[None, :]).astype(jnp.float32)
        q_ref[k, :, :] = lax.dot_general(ew, onehot, (((0,), (0,)), ((), ())),
                                         preferred_element_type=jnp.float32)
        part = part + jnp.sum(dmin, axis=(0, 1), keepdims=True)

    @pl.when(pid == 0)
    def _():
        loss_ref[...] = part

    @pl.when(pid != 0)
    def _():
        loss_ref[...] = loss_ref[...] + part


def kernel(inputs, embed_weight):
    input_shape = inputs.shape
    bb, cc, hh, ww = input_shape
    n = inputs.size
    nslab = bb * hh // 8
    xt = jnp.transpose(inputs, (0, 2, 3, 1))
    x3 = xt.reshape(nslab, 8 * ww, cc)
    x3 = lax.optimization_barrier(x3)
    grid = nslab // CB

    q, idx3, lsum = pl.pallas_call(
        _vq_kernel,
        grid=(grid,),
        in_specs=[
            pl.BlockSpec((CB, 8 * ww, cc), lambda i: (i, 0, 0)),
            pl.BlockSpec((N_E, D), lambda i: (0, 0)),
        ],
        out_specs=[
            pl.BlockSpec((CB, 8 * ww, cc), lambda i: (i, 0, 0)),
            pl.BlockSpec((CB, 1, cc), lambda i: (i, 0, 0)),
            pl.BlockSpec((1, 1), lambda i: (0, 0)),
        ],
        out_shape=[
            jax.ShapeDtypeStruct((nslab, 8 * ww, cc), jnp.float32),
            jax.ShapeDtypeStruct((nslab, 1, cc), jnp.int32),
            jax.ShapeDtypeStruct((1, 1), jnp.float32),
        ],
        scratch_shapes=[pltpu.VMEM((1, N_E), jnp.float32),
                        pltpu.VMEM((N_E, D), jnp.float32)],
        compiler_params=pltpu.CompilerParams(
            dimension_semantics=("arbitrary",)),
    )(x3, embed_weight)

    q4 = q.reshape(bb, hh, ww, cc)
    quantized = jnp.transpose(q4, (0, 3, 1, 2))
    loss = (lsum * jnp.float32((1.0 + BETA) / n)).reshape(())
    idxg = idx3.reshape(bb, hh // 8, cc)
    idx2d = jnp.transpose(idxg, (0, 2, 1)).reshape(bb, hh, ww)
    return (quantized, loss, idx2d)


# speedup vs baseline: 1.0368x; 1.0104x over previous
import jax
import jax.numpy as jnp
from jax import lax
from jax.experimental import pallas as pl
from jax.experimental.pallas import tpu as pltpu

N_E = 1024
D = 256
BETA = 0.25
CB = 8


def _vq_kernel(x_ref, e_ref, q_ref, idx_ref, loss_ref, e2_ref, em2_ref):
    pid = pl.program_id(0)
    ew = e_ref[...]

    @pl.when(pid == 0)
    def _():
        ones_col = jnp.ones((D, 1), jnp.float32)
        e2_ref[...] = lax.dot_general(
            ones_col, ew * ew, (((0,), (1,)), ((), ())),
            preferred_element_type=jnp.float32)
        em2_ref[...] = -2.0 * ew

    e2 = e2_ref[...]
    em2 = em2_ref[...]
    part = jnp.zeros((1, 1), jnp.float32)
    for k in range(CB):
        xk = x_ref[k]
        mm = lax.dot_general(xk.astype(jnp.bfloat16), em2,
                             (((0,), (1,)), ((), ())),
                             preferred_element_type=jnp.float32)
        x2 = lax.dot_general(xk * xk, jnp.ones((D, 1), jnp.float32),
                             (((0,), (0,)), ((), ())),
                             preferred_element_type=jnp.float32)
        d = (x2 + e2) + mm
        dmin = jnp.min(d, axis=1, keepdims=True)
        iota_j = lax.broadcasted_iota(jnp.int32, d.shape, 1)
        idx = jnp.min(jnp.where(d == dmin, iota_j, jnp.int32(2**30)), axis=1)
        idx_ref[k, 0, :] = idx
        onehot = (lax.broadcasted_iota(jnp.int32, (N_E, 256), 0)
                  == idx[None, :]).astype(jnp.bfloat16)
        q_ref[k, :, :] = lax.dot_general(ew.astype(jnp.bfloat16), onehot,
                                         (((0,), (0,)), ((), ())),
                                         preferred_element_type=jnp.float32)
        part = part + jnp.sum(dmin, axis=(0, 1), keepdims=True)

    @pl.when(pid == 0)
    def _():
        loss_ref[...] = part

    @pl.when(pid != 0)
    def _():
        loss_ref[...] = loss_ref[...] + part


def kernel(inputs, embed_weight):
    input_shape = inputs.shape
    bb, cc, hh, ww = input_shape
    n = inputs.size
    nslab = bb * hh // 8
    xt = jnp.transpose(inputs, (0, 2, 3, 1))
    x3 = xt.reshape(nslab, 8 * ww, cc)
    x3 = lax.optimization_barrier(x3)
    grid = nslab // CB

    q, idx3, lsum = pl.pallas_call(
        _vq_kernel,
        grid=(grid,),
        in_specs=[
            pl.BlockSpec((CB, 8 * ww, cc), lambda i: (i, 0, 0)),
            pl.BlockSpec((N_E, D), lambda i: (0, 0)),
        ],
        out_specs=[
            pl.BlockSpec((CB, 8 * ww, cc), lambda i: (i, 0, 0)),
            pl.BlockSpec((CB, 1, cc), lambda i: (i, 0, 0)),
            pl.BlockSpec((1, 1), lambda i: (0, 0)),
        ],
        out_shape=[
            jax.ShapeDtypeStruct((nslab, 8 * ww, cc), jnp.float32),
            jax.ShapeDtypeStruct((nslab, 1, cc), jnp.int32),
            jax.ShapeDtypeStruct((1, 1), jnp.float32),
        ],
        scratch_shapes=[pltpu.VMEM((1, N_E), jnp.float32),
                        pltpu.VMEM((N_E, D), jnp.float32)],
        compiler_params=pltpu.CompilerParams(
            dimension_semantics=("arbitrary",)),
    )(x3, embed_weight)

    q4 = q.reshape(bb, hh, ww, cc)
    quantized = jnp.transpose(q4, (0, 3, 1, 2))
    loss = (lsum * jnp.float32((1.0 + BETA) / n)).reshape(())
    idxg = idx3.reshape(bb, hh // 8, cc)
    idx2d = jnp.transpose(idxg, (0, 2, 1)).reshape(bb, hh, ww)
    return (quantized, loss, idx2d)
